# baseline (speedup 1.0000x reference)
import functools

import jax
import jax.numpy as jnp
from jax import lax
from jax.experimental import pallas as pl
from jax.experimental.pallas import tpu as pltpu
from jax.experimental.pallas import tpu_sc as plsc

N = 10000
NP = 10240
E = 320000
HID = 128
EDGE_DIM = 16
HID_E = 64
L = 5
FH = 12
OUT_FEAT = 4

NCORE = 2
NSUB = 16
NW = NCORE * NSUB
EW = 10240
E_PAD = NW * EW
C = 128
NCHUNK = EW // C
RPS = NP // NSUB

BE = 2048
BN = 512


def _edge_mlp_body(ea_ref, W1_ref, b1_ref, W2_ref, b2_ref, *w_refs):
    i = pl.program_id(0)
    ea = ea_ref[...]
    eid = i * BE + lax.broadcasted_iota(jnp.int32, (BE, 1), 0)
    valid = eid < E
    for l in range(L):
        t = jnp.maximum(ea @ W1_ref[l] + b1_ref[l][None, :], 0.0)
        s = t @ W2_ref[l] + b2_ref[l][None, :]
        w_refs[l][...] = jnp.where(valid, jax.nn.sigmoid(s), 0.0)[:, 0]


def _edge_mlp(ea_p, W1, b1, W2, b2):
    grid = (E_PAD // BE,)
    return pl.pallas_call(
        _edge_mlp_body,
        grid=grid,
        in_specs=[
            pl.BlockSpec((BE, EDGE_DIM), lambda i: (i, 0)),
            pl.BlockSpec((L, EDGE_DIM, HID_E), lambda i: (0, 0, 0)),
            pl.BlockSpec((L, HID_E), lambda i: (0, 0)),
            pl.BlockSpec((L, HID_E, 1), lambda i: (0, 0, 0)),
            pl.BlockSpec((L, 1), lambda i: (0, 0)),
        ],
        out_specs=[pl.BlockSpec((BE,), lambda i: (i,)) for _ in range(L)],
        out_shape=[jax.ShapeDtypeStruct((E_PAD,), jnp.float32)
                   for _ in range(L)],
    )(ea_p, W1, b1, W2, b2)


_DWIN = 8


def _deg_body(z_hbm, col_hbm, w0_h, w1_h, w2_h, w3_h, w4_h,
              o0, o1, o2, o3, o4,
              a0, a1, a2, a3, a4, colbuf, wb0, wb1, wb2, wb3, wb4, sem):
    ci = lax.axis_index("c")
    si = lax.axis_index("s")
    wid = ci * NSUB + si
    accs = (a0, a1, a2, a3, a4)
    wbufs = (wb0, wb1, wb2, wb3, wb4)
    whs = (w0_h, w1_h, w2_h, w3_h, w4_h)
    for l in range(L):
        pltpu.sync_copy(z_hbm, accs[l].at[pl.ds(si * RPS, RPS)])
        pltpu.sync_copy(whs[l].at[wid], wbufs[l])
    pltpu.sync_copy(col_hbm.at[wid], colbuf)
    plsc.subcore_barrier()

    def s_desc(k, l):
        return pltpu.make_async_copy(
            wbufs[l].at[k], accs[l].at[colbuf.at[k]], sem)

    for k in range(_DWIN):
        for l in range(L):
            s_desc(k, l).start(add=True)

    def chunk_body(k, _):
        for l in range(L):
            s_desc(k - _DWIN, l).wait()
            s_desc(k, l).start(add=True)
        return 0

    lax.fori_loop(_DWIN, NCHUNK, chunk_body, 0)
    for k in range(_DWIN):
        for l in range(L):
            s_desc(NCHUNK - _DWIN + k, l).wait()
    plsc.subcore_barrier()
    outs = (o0, o1, o2, o3, o4)
    for l in range(L):
        pltpu.sync_copy(accs[l].at[pl.ds(si * RPS, RPS)],
                        outs[l].at[ci, pl.ds(si * RPS, RPS)])


def _deg_partials(col3, wls):
    mesh = plsc.VectorSubcoreMesh(core_axis_name="c", subcore_axis_name="s")
    f = pl.kernel(
        _deg_body,
        out_type=[jax.ShapeDtypeStruct((NCORE, NP), jnp.float32)
                  for _ in range(L)],
        mesh=mesh,
        scratch_types=(
            [pltpu.VMEM_SHARED((NP,), jnp.float32) for _ in range(L)]
            + [pltpu.VMEM((NCHUNK, C), jnp.int32)]
            + [pltpu.VMEM((NCHUNK, C), jnp.float32) for _ in range(L)]
            + [pltpu.SemaphoreType.DMA]
        ),
    )
    return f(jnp.zeros((RPS,), jnp.float32), col3, *wls)


def _embed_body(x_ref, eW_ref, eb_ref, g_ref, b_ref,
                d0_ref, d1_ref, d2_ref, d3_ref, d4_ref, W0_ref,
                dis_ref, xls_ref):
    h = x_ref[...] @ eW_ref[...] + eb_ref[...][None, :]
    mu = jnp.mean(h, axis=-1, keepdims=True)
    var = jnp.mean((h - mu) ** 2, axis=-1, keepdims=True)
    h = (h - mu) * lax.rsqrt(var + 1e-5) * g_ref[...][None, :] + b_ref[...][None, :]
    h = jnp.maximum(h, 0.0)
    drefs = (d0_ref, d1_ref, d2_ref, d3_ref, d4_ref)
    dis = jnp.concatenate(
        [lax.rsqrt(1.0 + dr[0] + dr[1])[None, :] for dr in drefs], axis=0)
    dis_ref[...] = dis
    xls_ref[...] = dis[0][:, None] * (h @ W0_ref[...])


def _embed(x2, emb_W, emb_b, ln_g, ln_b, degp, lin_W0):
    grid = (NP // BN,)
    return pl.pallas_call(
        _embed_body,
        grid=grid,
        in_specs=[
            pl.BlockSpec((BN, EDGE_DIM * 3), lambda i: (i, 0)),
            pl.BlockSpec((EDGE_DIM * 3, HID), lambda i: (0, 0)),
            pl.BlockSpec((HID,), lambda i: (0,)),
            pl.BlockSpec((HID,), lambda i: (0,)),
            pl.BlockSpec((HID,), lambda i: (0,)),
        ] + [pl.BlockSpec((NCORE, BN), lambda i: (0, i)) for _ in range(L)] + [
            pl.BlockSpec((HID, HID), lambda i: (0, 0)),
        ],
        out_specs=[
            pl.BlockSpec((L, BN), lambda i: (0, i)),
            pl.BlockSpec((BN, HID), lambda i: (i, 0)),
        ],
        out_shape=[
            jax.ShapeDtypeStruct((L, NP), jnp.float32),
            jax.ShapeDtypeStruct((NP, HID), jnp.float32),
        ],
    )(x2, emb_W, emb_b, ln_g, ln_b, *degp, lin_W0)


NB = 2
SEG = 16
NSEG = NCHUNK // SEG


def _msg_body(xls_hbm, row4, col4, w4, out_hbm,
              acc, b0, b1, rowbuf, colbuf, wbuf,
              sg0, sg1, ss0, ss1):
    ci = lax.axis_index("c")
    si = lax.axis_index("s")
    wid = ci * NSUB + si
    bufs = (b0, b1)
    sgs = (sg0, sg1)
    sss = (ss0, ss1)
    zeros = jnp.zeros((16,), jnp.float32)

    def zrow(i, _):
        for j in range(HID // 16):
            b0[i, pl.ds(j * 16, 16)] = zeros
        return 0

    lax.fori_loop(0, C, zrow, 0)
    for k in range(RPS // C):
        pltpu.sync_copy(b0, acc.at[pl.ds(si * RPS + k * C, C)])
    plsc.subcore_barrier()

    def g_desc(j, i):
        return pltpu.make_async_copy(xls_hbm.at[rowbuf.at[j]], bufs[i], sgs[i])

    def s_desc(j, i):
        return pltpu.make_async_copy(bufs[i], acc.at[colbuf.at[j]], sss[i])

    def scale(j, i):
        buf = bufs[i]

        def g(t, _):
            wv = wbuf[j, pl.ds(t * 16, 16)]
            for kk in range(16):
                we = wv[kk]
                e = t * 16 + kk
                for jj in range(HID // 16):
                    sl = pl.ds(jj * 16, 16)
                    buf[e, sl] = buf[e, sl] * we
            return 0

        lax.fori_loop(0, C // 16, g, 0)

    def seg_body(s, _):
        pltpu.sync_copy(row4.at[wid, s], rowbuf)
        pltpu.sync_copy(col4.at[wid, s], colbuf)
        pltpu.sync_copy(w4.at[wid, s], wbuf)
        for i in range(NB):
            g_desc(i, i).start()

        def body(k2, _):
            js = [k2 * NB + i for i in range(NB)]
            for i in range(NB):
                g_desc(js[i], i).wait()
                scale(js[i], i)
                s_desc(js[i], i).start(add=True)
            for i in range(NB):
                s_desc(js[i], i).wait()

                @pl.when(js[i] + NB < SEG)
                def _():
                    g_desc(js[i] + NB, i).start()
            return 0

        lax.fori_loop(0, SEG // NB, body, 0)
        return 0

    lax.fori_loop(0, NSEG, seg_body, 0)
    plsc.subcore_barrier()
    pltpu.sync_copy(acc.at[pl.ds(si * RPS, RPS)],
                    out_hbm.at[ci, pl.ds(si * RPS, RPS)])


def _msg_pass(xls, row4, col4, wl4):
    mesh = plsc.VectorSubcoreMesh(core_axis_name="c", subcore_axis_name="s")
    f = pl.kernel(
        _msg_body,
        out_type=jax.ShapeDtypeStruct((NCORE, NP, HID), jnp.float32),
        mesh=mesh,
        scratch_types=(
            [pltpu.VMEM_SHARED((NP, HID), jnp.float32)]
            + [pltpu.VMEM((C, HID), jnp.float32) for _ in range(NB)]
            + [pltpu.VMEM((SEG, C), jnp.int32)] * 2
            + [pltpu.VMEM((SEG, C), jnp.float32)]
            + [pltpu.SemaphoreType.DMA] * (2 * NB)
        ),
    )
    return f(xls, row4, col4, wl4)


def _mid_body(p_ref, xls_ref, dis_ref, bias_ref, W_ref, out_ref, *, l):
    ssum = p_ref[0] + p_ref[1] + xls_ref[...]
    dis = dis_ref[...]
    h = jnp.maximum(dis[l][:, None] * ssum + bias_ref[l][None, :], 0.0)
    out_ref[...] = dis[l + 1][:, None] * (h @ W_ref[0])


def _mid(p, xls, dis, conv_bias, lin_W, l):
    grid = (NP // BN,)
    return pl.pallas_call(
        functools.partial(_mid_body, l=l),
        grid=grid,
        in_specs=[
            pl.BlockSpec((NCORE, BN, HID), lambda i: (0, i, 0)),
            pl.BlockSpec((BN, HID), lambda i: (i, 0)),
            pl.BlockSpec((L, BN), lambda i: (0, i)),
            pl.BlockSpec((L, HID), lambda i: (0, 0)),
            pl.BlockSpec((1, HID, HID), lambda i: (l + 1, 0, 0)),
        ],
        out_specs=pl.BlockSpec((BN, HID), lambda i: (i, 0)),
        out_shape=jax.ShapeDtypeStruct((NP, HID), jnp.float32),
    )(p, xls, dis, conv_bias, lin_W)


def _final_body(p_ref, xls_ref, dis_ref, bias_ref, dW_ref, db_ref, out_ref):
    ssum = p_ref[0] + p_ref[1] + xls_ref[...]
    dis = dis_ref[...]
    h = jnp.maximum(dis[L - 1][:, None] * ssum + bias_ref[L - 1][None, :], 0.0)
    out_ref[...] = h @ dW_ref[...] + db_ref[...][None, :]


def _final(p, xls, dis, conv_bias, dec_W, dec_b):
    grid = (NP // BN,)
    DO = OUT_FEAT * FH
    return pl.pallas_call(
        _final_body,
        grid=grid,
        in_specs=[
            pl.BlockSpec((NCORE, BN, HID), lambda i: (0, i, 0)),
            pl.BlockSpec((BN, HID), lambda i: (i, 0)),
            pl.BlockSpec((L, BN), lambda i: (0, i)),
            pl.BlockSpec((L, HID), lambda i: (0, 0)),
            pl.BlockSpec((HID, DO), lambda i: (0, 0)),
            pl.BlockSpec((DO,), lambda i: (0,)),
        ],
        out_specs=pl.BlockSpec((BN, DO), lambda i: (i, 0)),
        out_shape=jax.ShapeDtypeStruct((NP, DO), jnp.float32),
    )(p, xls, dis, conv_bias, dec_W, dec_b)


def kernel(x, edge_index, edge_attr, emb_W, emb_b, ln_g, ln_b, lin_W,
           mlp_W1, mlp_b1, mlp_W2, mlp_b2, conv_bias, dec_W, dec_b):
    x2 = jnp.pad(x.reshape(N, -1), ((0, NP - N), (0, 0)))
    pad = E_PAD - E
    row_p = jnp.concatenate([edge_index[0], jnp.zeros((pad,), edge_index.dtype)])
    col_p = jnp.concatenate([edge_index[1], jnp.zeros((pad,), edge_index.dtype)])

    ea_p = jnp.concatenate([edge_attr, jnp.zeros((pad, EDGE_DIM), edge_attr.dtype)])
    ws = _edge_mlp(ea_p, mlp_W1, mlp_b1, mlp_W2, mlp_b2)
    col3 = col_p.reshape(NW, NCHUNK, C)
    wls = [wl.reshape(NW, NCHUNK, C) for wl in ws]
    degp = _deg_partials(col3, wls)
    row4 = row_p.reshape(NW, NSEG, SEG, C)
    col4 = col_p.reshape(NW, NSEG, SEG, C)
    wl4s = [wl.reshape(NW, NSEG, SEG, C) for wl in wls]
    dis, xls = _embed(x2, emb_W, emb_b, ln_g, ln_b, degp, lin_W[0])
    for l in range(L):
        p = _msg_pass(xls, row4, col4, wl4s[l])
        if l < L - 1:
            xls = _mid(p, xls, dis, conv_bias, lin_W, l)
        else:
            out = _final(p, xls, dis, conv_bias, dec_W, dec_b)
    return out[:N].reshape(N, OUT_FEAT, FH)


# speedup vs baseline: 1.1399x; 1.1399x over previous
import functools

import jax
import jax.numpy as jnp
from jax import lax
from jax.experimental import pallas as pl
from jax.experimental.pallas import tpu as pltpu
from jax.experimental.pallas import tpu_sc as plsc

N = 10000
NP = 10240
E = 320000
HID = 128
EDGE_DIM = 16
HID_E = 64
L = 5
FH = 12
OUT_FEAT = 4

NCORE = 2
NSUB = 16
NW = NCORE * NSUB
EW = 10240
E_PAD = NW * EW
C = 128
NCHUNK = EW // C
RPS = NP // NSUB

BE = 2048
BN = 512


def _edge_mlp_body(ea_ref, W1_ref, b1_ref, W2_ref, b2_ref, *w_refs):
    i = pl.program_id(0)
    ea = ea_ref[...]
    eid = i * BE + lax.broadcasted_iota(jnp.int32, (BE, 1), 0)
    valid = eid < E
    w_ref = w_refs[0]
    rows = []
    for l in range(L):
        t = jnp.maximum(ea @ W1_ref[l] + b1_ref[l][None, :], 0.0)
        s = t @ W2_ref[l] + b2_ref[l][None, :]
        rows.append(jnp.where(valid, jax.nn.sigmoid(s), 0.0)[:, 0][None, :])
    w_ref[...] = jnp.concatenate(rows, axis=0)


def _edge_mlp(ea_p, W1, b1, W2, b2):
    grid = (E_PAD // BE,)
    return pl.pallas_call(
        _edge_mlp_body,
        grid=grid,
        in_specs=[
            pl.BlockSpec((BE, EDGE_DIM), lambda i: (i, 0)),
            pl.BlockSpec((L, EDGE_DIM, HID_E), lambda i: (0, 0, 0)),
            pl.BlockSpec((L, HID_E), lambda i: (0, 0)),
            pl.BlockSpec((L, HID_E, 1), lambda i: (0, 0, 0)),
            pl.BlockSpec((L, 1), lambda i: (0, 0)),
        ],
        out_specs=pl.BlockSpec((L, BE), lambda i: (0, i)),
        out_shape=jax.ShapeDtypeStruct((L, E_PAD), jnp.float32),
    )(ea_p, W1, b1, W2, b2)


_DWIN = 8


def _deg_body(z_hbm, col_hbm, w0_h, w1_h, w2_h, w3_h, w4_h,
              o0, o1, o2, o3, o4,
              a0, a1, a2, a3, a4, colbuf, wb0, wb1, wb2, wb3, wb4, sem):
    ci = lax.axis_index("c")
    si = lax.axis_index("s")
    wid = ci * NSUB + si
    accs = (a0, a1, a2, a3, a4)
    wbufs = (wb0, wb1, wb2, wb3, wb4)
    whs = (w0_h, w1_h, w2_h, w3_h, w4_h)
    for l in range(L):
        pltpu.sync_copy(z_hbm, accs[l].at[pl.ds(si * RPS, RPS)])
        pltpu.sync_copy(whs[l].at[wid], wbufs[l])
    pltpu.sync_copy(col_hbm.at[wid], colbuf)
    plsc.subcore_barrier()

    def s_desc(k, l):
        return pltpu.make_async_copy(
            wbufs[l].at[k], accs[l].at[colbuf.at[k]], sem)

    for k in range(_DWIN):
        for l in range(L):
            s_desc(k, l).start(add=True)

    def chunk_body(k, _):
        for l in range(L):
            s_desc(k - _DWIN, l).wait()
            s_desc(k, l).start(add=True)
        return 0

    lax.fori_loop(_DWIN, NCHUNK, chunk_body, 0)
    for k in range(_DWIN):
        for l in range(L):
            s_desc(NCHUNK - _DWIN + k, l).wait()
    plsc.subcore_barrier()
    outs = (o0, o1, o2, o3, o4)
    for l in range(L):
        pltpu.sync_copy(accs[l].at[pl.ds(si * RPS, RPS)],
                        outs[l].at[ci, pl.ds(si * RPS, RPS)])


def _deg_partials(col3, wls):
    mesh = plsc.VectorSubcoreMesh(core_axis_name="c", subcore_axis_name="s")
    f = pl.kernel(
        _deg_body,
        out_type=[jax.ShapeDtypeStruct((NCORE, NP), jnp.float32)
                  for _ in range(L)],
        mesh=mesh,
        scratch_types=(
            [pltpu.VMEM_SHARED((NP,), jnp.float32) for _ in range(L)]
            + [pltpu.VMEM((NCHUNK, C), jnp.int32)]
            + [pltpu.VMEM((NCHUNK, C), jnp.float32) for _ in range(L)]
            + [pltpu.SemaphoreType.DMA]
        ),
    )
    return f(jnp.zeros((RPS,), jnp.float32), col3, *wls)


def _embed_body(x_ref, eW_ref, eb_ref, g_ref, b_ref,
                d0_ref, d1_ref, d2_ref, d3_ref, d4_ref, W0_ref,
                dis_ref, xls_ref):
    h = x_ref[...] @ eW_ref[...] + eb_ref[...][None, :]
    mu = jnp.mean(h, axis=-1, keepdims=True)
    var = jnp.mean((h - mu) ** 2, axis=-1, keepdims=True)
    h = (h - mu) * lax.rsqrt(var + 1e-5) * g_ref[...][None, :] + b_ref[...][None, :]
    h = jnp.maximum(h, 0.0)
    drefs = (d0_ref, d1_ref, d2_ref, d3_ref, d4_ref)
    dis = jnp.concatenate(
        [lax.rsqrt(1.0 + dr[0] + dr[1])[None, :] for dr in drefs], axis=0)
    dis_ref[...] = dis
    xls_ref[...] = dis[0][:, None] * (h @ W0_ref[...])


def _embed(x2, emb_W, emb_b, ln_g, ln_b, degp, lin_W0):
    grid = (NP // BN,)
    return pl.pallas_call(
        _embed_body,
        grid=grid,
        in_specs=[
            pl.BlockSpec((BN, EDGE_DIM * 3), lambda i: (i, 0)),
            pl.BlockSpec((EDGE_DIM * 3, HID), lambda i: (0, 0)),
            pl.BlockSpec((HID,), lambda i: (0,)),
            pl.BlockSpec((HID,), lambda i: (0,)),
            pl.BlockSpec((HID,), lambda i: (0,)),
        ] + [pl.BlockSpec((NCORE, BN), lambda i: (0, i)) for _ in range(L)] + [
            pl.BlockSpec((HID, HID), lambda i: (0, 0)),
        ],
        out_specs=[
            pl.BlockSpec((L, BN), lambda i: (0, i)),
            pl.BlockSpec((BN, HID), lambda i: (i, 0)),
        ],
        out_shape=[
            jax.ShapeDtypeStruct((L, NP), jnp.float32),
            jax.ShapeDtypeStruct((NP, HID), jnp.float32),
        ],
    )(x2, emb_W, emb_b, ln_g, ln_b, *degp, lin_W0)


NB = 2
SEG = 16
NSEG = NCHUNK // SEG


def _msg_body(xls_hbm, row4, col4, w4, out_hbm,
              acc, b0, b1, rowbuf, colbuf, wbuf,
              sg0, sg1, ss0, ss1):
    ci = lax.axis_index("c")
    si = lax.axis_index("s")
    wid = ci * NSUB + si
    bufs = (b0, b1)
    sgs = (sg0, sg1)
    sss = (ss0, ss1)
    zeros = jnp.zeros((16,), jnp.float32)

    def zrow(i, _):
        for j in range(HID // 16):
            b0[i, pl.ds(j * 16, 16)] = zeros
        return 0

    lax.fori_loop(0, C, zrow, 0)
    for k in range(RPS // C):
        pltpu.sync_copy(b0, acc.at[pl.ds(si * RPS + k * C, C)])
    plsc.subcore_barrier()

    def g_desc(j, i):
        return pltpu.make_async_copy(xls_hbm.at[rowbuf.at[j]], bufs[i], sgs[i])

    def s_desc(j, i):
        return pltpu.make_async_copy(bufs[i], acc.at[colbuf.at[j]], sss[i])

    def scale(j, i):
        buf = bufs[i]

        def g(t, _):
            wv = wbuf[j, pl.ds(t * 16, 16)]
            for kk in range(16):
                we = wv[kk]
                e = t * 16 + kk
                for jj in range(HID // 16):
                    sl = pl.ds(jj * 16, 16)
                    buf[e, sl] = buf[e, sl] * we
            return 0

        lax.fori_loop(0, C // 16, g, 0)

    def seg_body(s, _):
        pltpu.sync_copy(row4.at[wid, s], rowbuf)
        pltpu.sync_copy(col4.at[wid, s], colbuf)
        pltpu.sync_copy(w4.at[wid, s], wbuf)
        for i in range(NB):
            g_desc(i, i).start()

        def body(k2, _):
            js = [k2 * NB + i for i in range(NB)]
            for i in range(NB):
                g_desc(js[i], i).wait()
                scale(js[i], i)
                s_desc(js[i], i).start(add=True)
            for i in range(NB):
                s_desc(js[i], i).wait()

                @pl.when(js[i] + NB < SEG)
                def _():
                    g_desc(js[i] + NB, i).start()
            return 0

        lax.fori_loop(0, SEG // NB, body, 0)
        return 0

    lax.fori_loop(0, NSEG, seg_body, 0)
    plsc.subcore_barrier()
    pltpu.sync_copy(acc.at[pl.ds(si * RPS, RPS)],
                    out_hbm.at[ci, pl.ds(si * RPS, RPS)])


def _msg_pass(xls, row4, col4, wl4):
    mesh = plsc.VectorSubcoreMesh(core_axis_name="c", subcore_axis_name="s")
    f = pl.kernel(
        _msg_body,
        out_type=jax.ShapeDtypeStruct((NCORE, NP, HID), jnp.float32),
        mesh=mesh,
        scratch_types=(
            [pltpu.VMEM_SHARED((NP, HID), jnp.float32)]
            + [pltpu.VMEM((C, HID), jnp.float32) for _ in range(NB)]
            + [pltpu.VMEM((SEG, C), jnp.int32)] * 2
            + [pltpu.VMEM((SEG, C), jnp.float32)]
            + [pltpu.SemaphoreType.DMA] * (2 * NB)
        ),
    )
    return f(xls, row4, col4, wl4)


def _mid_body(p_ref, xls_ref, dis_ref, bias_ref, W_ref, out_ref, *, l):
    ssum = p_ref[0] + p_ref[1] + xls_ref[...]
    dis = dis_ref[...]
    h = jnp.maximum(dis[l][:, None] * ssum + bias_ref[l][None, :], 0.0)
    out_ref[...] = dis[l + 1][:, None] * (h @ W_ref[0])


def _mid(p, xls, dis, conv_bias, lin_W, l):
    grid = (NP // BN,)
    return pl.pallas_call(
        functools.partial(_mid_body, l=l),
        grid=grid,
        in_specs=[
            pl.BlockSpec((NCORE, BN, HID), lambda i: (0, i, 0)),
            pl.BlockSpec((BN, HID), lambda i: (i, 0)),
            pl.BlockSpec((L, BN), lambda i: (0, i)),
            pl.BlockSpec((L, HID), lambda i: (0, 0)),
            pl.BlockSpec((1, HID, HID), lambda i: (l + 1, 0, 0)),
        ],
        out_specs=pl.BlockSpec((BN, HID), lambda i: (i, 0)),
        out_shape=jax.ShapeDtypeStruct((NP, HID), jnp.float32),
    )(p, xls, dis, conv_bias, lin_W)


def _final_body(p_ref, xls_ref, dis_ref, bias_ref, dW_ref, db_ref, out_ref):
    ssum = p_ref[0] + p_ref[1] + xls_ref[...]
    dis = dis_ref[...]
    h = jnp.maximum(dis[L - 1][:, None] * ssum + bias_ref[L - 1][None, :], 0.0)
    out_ref[...] = h @ dW_ref[...] + db_ref[...][None, :]


def _final(p, xls, dis, conv_bias, dec_W, dec_b):
    grid = (NP // BN,)
    DO = OUT_FEAT * FH
    return pl.pallas_call(
        _final_body,
        grid=grid,
        in_specs=[
            pl.BlockSpec((NCORE, BN, HID), lambda i: (0, i, 0)),
            pl.BlockSpec((BN, HID), lambda i: (i, 0)),
            pl.BlockSpec((L, BN), lambda i: (0, i)),
            pl.BlockSpec((L, HID), lambda i: (0, 0)),
            pl.BlockSpec((HID, DO), lambda i: (0, 0)),
            pl.BlockSpec((DO,), lambda i: (0,)),
        ],
        out_specs=pl.BlockSpec((BN, DO), lambda i: (i, 0)),
        out_shape=jax.ShapeDtypeStruct((NP, DO), jnp.float32),
    )(p, xls, dis, conv_bias, dec_W, dec_b)


def kernel(x, edge_index, edge_attr, emb_W, emb_b, ln_g, ln_b, lin_W,
           mlp_W1, mlp_b1, mlp_W2, mlp_b2, conv_bias, dec_W, dec_b):
    x2 = jnp.pad(x.reshape(N, -1), ((0, NP - N), (0, 0)))
    pad = E_PAD - E
    row_p = jnp.concatenate([edge_index[0], jnp.zeros((pad,), edge_index.dtype)])
    col_p = jnp.concatenate([edge_index[1], jnp.zeros((pad,), edge_index.dtype)])

    ea_p = jnp.concatenate([edge_attr, jnp.zeros((pad, EDGE_DIM), edge_attr.dtype)])
    w = _edge_mlp(ea_p, mlp_W1, mlp_b1, mlp_W2, mlp_b2)
    col3 = col_p.reshape(NW, NCHUNK, C)
    wls = [w[l].reshape(NW, NCHUNK, C) for l in range(L)]
    degp = _deg_partials(col3, wls)
    row4 = row_p.reshape(NW, NSEG, SEG, C)
    col4 = col_p.reshape(NW, NSEG, SEG, C)
    wl4s = [wl.reshape(NW, NSEG, SEG, C) for wl in wls]
    dis, xls = _embed(x2, emb_W, emb_b, ln_g, ln_b, degp, lin_W[0])
    for l in range(L):
        p = _msg_pass(xls, row4, col4, wl4s[l])
        if l < L - 1:
            xls = _mid(p, xls, dis, conv_bias, lin_W, l)
        else:
            out = _final(p, xls, dis, conv_bias, dec_W, dec_b)
    return out[:N].reshape(N, OUT_FEAT, FH)


# speedup vs baseline: 1.1724x; 1.0286x over previous
import functools

import jax
import jax.numpy as jnp
from jax import lax
from jax.experimental import pallas as pl
from jax.experimental.pallas import tpu as pltpu
from jax.experimental.pallas import tpu_sc as plsc

N = 10000
NP = 10240
E = 320000
HID = 128
EDGE_DIM = 16
HID_E = 64
L = 5
FH = 12
OUT_FEAT = 4

NCORE = 2
NSUB = 16
NW = NCORE * NSUB
EW = 10240
E_PAD = NW * EW
C = 64
NCHUNK = EW // C
RPS = NP // NSUB

BE = 2048
BN = 512


def _edge_mlp_body(ea_ref, W1_ref, b1_ref, W2_ref, b2_ref, *w_refs):
    i = pl.program_id(0)
    ea = ea_ref[...]
    eid = i * BE + lax.broadcasted_iota(jnp.int32, (BE, 1), 0)
    valid = eid < E
    w_ref = w_refs[0]
    rows = []
    for l in range(L):
        t = jnp.maximum(ea @ W1_ref[l] + b1_ref[l][None, :], 0.0)
        s = t @ W2_ref[l] + b2_ref[l][None, :]
        rows.append(jnp.where(valid, jax.nn.sigmoid(s), 0.0)[:, 0][None, :])
    w_ref[...] = jnp.concatenate(rows, axis=0)


def _edge_mlp(ea_p, W1, b1, W2, b2):
    grid = (E_PAD // BE,)
    return pl.pallas_call(
        _edge_mlp_body,
        grid=grid,
        in_specs=[
            pl.BlockSpec((BE, EDGE_DIM), lambda i: (i, 0)),
            pl.BlockSpec((L, EDGE_DIM, HID_E), lambda i: (0, 0, 0)),
            pl.BlockSpec((L, HID_E), lambda i: (0, 0)),
            pl.BlockSpec((L, HID_E, 1), lambda i: (0, 0, 0)),
            pl.BlockSpec((L, 1), lambda i: (0, 0)),
        ],
        out_specs=pl.BlockSpec((L, BE), lambda i: (0, i)),
        out_shape=jax.ShapeDtypeStruct((L, E_PAD), jnp.float32),
    )(ea_p, W1, b1, W2, b2)


_DWIN = 8


def _deg_body(z_hbm, col_hbm, w0_h, w1_h, w2_h, w3_h, w4_h,
              o0, o1, o2, o3, o4,
              a0, a1, a2, a3, a4, colbuf, wb0, wb1, wb2, wb3, wb4, sem):
    ci = lax.axis_index("c")
    si = lax.axis_index("s")
    wid = ci * NSUB + si
    accs = (a0, a1, a2, a3, a4)
    wbufs = (wb0, wb1, wb2, wb3, wb4)
    whs = (w0_h, w1_h, w2_h, w3_h, w4_h)
    for l in range(L):
        pltpu.sync_copy(z_hbm, accs[l].at[pl.ds(si * RPS, RPS)])
        pltpu.sync_copy(whs[l].at[wid], wbufs[l])
    pltpu.sync_copy(col_hbm.at[wid], colbuf)
    plsc.subcore_barrier()

    def s_desc(k, l):
        return pltpu.make_async_copy(
            wbufs[l].at[k], accs[l].at[colbuf.at[k]], sem)

    for k in range(_DWIN):
        for l in range(L):
            s_desc(k, l).start(add=True)

    def chunk_body(k, _):
        for l in range(L):
            s_desc(k - _DWIN, l).wait()
            s_desc(k, l).start(add=True)
        return 0

    lax.fori_loop(_DWIN, NCHUNK, chunk_body, 0)
    for k in range(_DWIN):
        for l in range(L):
            s_desc(NCHUNK - _DWIN + k, l).wait()
    plsc.subcore_barrier()
    outs = (o0, o1, o2, o3, o4)
    for l in range(L):
        pltpu.sync_copy(accs[l].at[pl.ds(si * RPS, RPS)],
                        outs[l].at[ci, pl.ds(si * RPS, RPS)])


def _deg_partials(col3, wls):
    mesh = plsc.VectorSubcoreMesh(core_axis_name="c", subcore_axis_name="s")
    f = pl.kernel(
        _deg_body,
        out_type=[jax.ShapeDtypeStruct((NCORE, NP), jnp.float32)
                  for _ in range(L)],
        mesh=mesh,
        scratch_types=(
            [pltpu.VMEM_SHARED((NP,), jnp.float32) for _ in range(L)]
            + [pltpu.VMEM((NCHUNK, C), jnp.int32)]
            + [pltpu.VMEM((NCHUNK, C), jnp.float32) for _ in range(L)]
            + [pltpu.SemaphoreType.DMA]
        ),
    )
    return f(jnp.zeros((RPS,), jnp.float32), col3, *wls)


def _embed_body(x_ref, eW_ref, eb_ref, g_ref, b_ref,
                d0_ref, d1_ref, d2_ref, d3_ref, d4_ref, W0_ref,
                dis_ref, xls_ref):
    h = x_ref[...] @ eW_ref[...] + eb_ref[...][None, :]
    mu = jnp.mean(h, axis=-1, keepdims=True)
    var = jnp.mean((h - mu) ** 2, axis=-1, keepdims=True)
    h = (h - mu) * lax.rsqrt(var + 1e-5) * g_ref[...][None, :] + b_ref[...][None, :]
    h = jnp.maximum(h, 0.0)
    drefs = (d0_ref, d1_ref, d2_ref, d3_ref, d4_ref)
    dis = jnp.concatenate(
        [lax.rsqrt(1.0 + dr[0] + dr[1])[None, :] for dr in drefs], axis=0)
    dis_ref[...] = dis
    xls_ref[...] = dis[0][:, None] * (h @ W0_ref[...])


def _embed(x2, emb_W, emb_b, ln_g, ln_b, degp, lin_W0):
    grid = (NP // BN,)
    return pl.pallas_call(
        _embed_body,
        grid=grid,
        in_specs=[
            pl.BlockSpec((BN, EDGE_DIM * 3), lambda i: (i, 0)),
            pl.BlockSpec((EDGE_DIM * 3, HID), lambda i: (0, 0)),
            pl.BlockSpec((HID,), lambda i: (0,)),
            pl.BlockSpec((HID,), lambda i: (0,)),
            pl.BlockSpec((HID,), lambda i: (0,)),
        ] + [pl.BlockSpec((NCORE, BN), lambda i: (0, i)) for _ in range(L)] + [
            pl.BlockSpec((HID, HID), lambda i: (0, 0)),
        ],
        out_specs=[
            pl.BlockSpec((L, BN), lambda i: (0, i)),
            pl.BlockSpec((BN, HID), lambda i: (i, 0)),
        ],
        out_shape=[
            jax.ShapeDtypeStruct((L, NP), jnp.float32),
            jax.ShapeDtypeStruct((NP, HID), jnp.float32),
        ],
    )(x2, emb_W, emb_b, ln_g, ln_b, *degp, lin_W0)


NB = 4
SEG = 16
NSEG = NCHUNK // SEG


def _msg_body(xls_hbm, row4, col4, w4, out_hbm,
              acc, b0, b1, b2, b3, rowbuf, colbuf, wbuf,
              sg0, sg1, sg2, sg3, ss0, ss1, ss2, ss3):
    ci = lax.axis_index("c")
    si = lax.axis_index("s")
    wid = ci * NSUB + si
    bufs = (b0, b1, b2, b3)
    sgs = (sg0, sg1, sg2, sg3)
    sss = (ss0, ss1, ss2, ss3)
    zeros = jnp.zeros((16,), jnp.float32)

    def zrow(i, _):
        for j in range(HID // 16):
            b0[i, pl.ds(j * 16, 16)] = zeros
        return 0

    lax.fori_loop(0, C, zrow, 0)
    for k in range(RPS // C):
        pltpu.sync_copy(b0, acc.at[pl.ds(si * RPS + k * C, C)])
    plsc.subcore_barrier()

    def g_desc(j, i):
        return pltpu.make_async_copy(xls_hbm.at[rowbuf.at[j]], bufs[i], sgs[i])

    def s_desc(j, i):
        return pltpu.make_async_copy(bufs[i], acc.at[colbuf.at[j]], sss[i])

    def scale(j, i):
        buf = bufs[i]

        def g(t, _):
            wv = wbuf[j, pl.ds(t * 16, 16)]
            for kk in range(16):
                we = wv[kk]
                e = t * 16 + kk
                for jj in range(HID // 16):
                    sl = pl.ds(jj * 16, 16)
                    buf[e, sl] = buf[e, sl] * we
            return 0

        lax.fori_loop(0, C // 16, g, 0)

    def seg_body(s, _):
        pltpu.sync_copy(row4.at[wid, s], rowbuf)
        pltpu.sync_copy(col4.at[wid, s], colbuf)
        pltpu.sync_copy(w4.at[wid, s], wbuf)
        for i in range(NB):
            g_desc(i, i).start()

        def body(k2, _):
            js = [k2 * NB + i for i in range(NB)]
            for i in range(NB):
                g_desc(js[i], i).wait()
                scale(js[i], i)
                s_desc(js[i], i).start(add=True)
            for i in range(NB):
                s_desc(js[i], i).wait()

                @pl.when(js[i] + NB < SEG)
                def _():
                    g_desc(js[i] + NB, i).start()
            return 0

        lax.fori_loop(0, SEG // NB, body, 0)
        return 0

    lax.fori_loop(0, NSEG, seg_body, 0)
    plsc.subcore_barrier()
    pltpu.sync_copy(acc.at[pl.ds(si * RPS, RPS)],
                    out_hbm.at[ci, pl.ds(si * RPS, RPS)])


def _msg_pass(xls, row4, col4, wl4):
    mesh = plsc.VectorSubcoreMesh(core_axis_name="c", subcore_axis_name="s")
    f = pl.kernel(
        _msg_body,
        out_type=jax.ShapeDtypeStruct((NCORE, NP, HID), jnp.float32),
        mesh=mesh,
        scratch_types=(
            [pltpu.VMEM_SHARED((NP, HID), jnp.float32)]
            + [pltpu.VMEM((C, HID), jnp.float32) for _ in range(NB)]
            + [pltpu.VMEM((SEG, C), jnp.int32)] * 2
            + [pltpu.VMEM((SEG, C), jnp.float32)]
            + [pltpu.SemaphoreType.DMA] * (2 * NB)
        ),
    )
    return f(xls, row4, col4, wl4)


def _mid_body(p_ref, xls_ref, dis_ref, bias_ref, W_ref, out_ref, *, l):
    ssum = p_ref[0] + p_ref[1] + xls_ref[...]
    dis = dis_ref[...]
    h = jnp.maximum(dis[l][:, None] * ssum + bias_ref[l][None, :], 0.0)
    out_ref[...] = dis[l + 1][:, None] * (h @ W_ref[0])


def _mid(p, xls, dis, conv_bias, lin_W, l):
    grid = (NP // BN,)
    return pl.pallas_call(
        functools.partial(_mid_body, l=l),
        grid=grid,
        in_specs=[
            pl.BlockSpec((NCORE, BN, HID), lambda i: (0, i, 0)),
            pl.BlockSpec((BN, HID), lambda i: (i, 0)),
            pl.BlockSpec((L, BN), lambda i: (0, i)),
            pl.BlockSpec((L, HID), lambda i: (0, 0)),
            pl.BlockSpec((1, HID, HID), lambda i: (l + 1, 0, 0)),
        ],
        out_specs=pl.BlockSpec((BN, HID), lambda i: (i, 0)),
        out_shape=jax.ShapeDtypeStruct((NP, HID), jnp.float32),
    )(p, xls, dis, conv_bias, lin_W)


def _final_body(p_ref, xls_ref, dis_ref, bias_ref, dW_ref, db_ref, out_ref):
    ssum = p_ref[0] + p_ref[1] + xls_ref[...]
    dis = dis_ref[...]
    h = jnp.maximum(dis[L - 1][:, None] * ssum + bias_ref[L - 1][None, :], 0.0)
    out_ref[...] = h @ dW_ref[...] + db_ref[...][None, :]


def _final(p, xls, dis, conv_bias, dec_W, dec_b):
    grid = (NP // BN,)
    DO = OUT_FEAT * FH
    return pl.pallas_call(
        _final_body,
        grid=grid,
        in_specs=[
            pl.BlockSpec((NCORE, BN, HID), lambda i: (0, i, 0)),
            pl.BlockSpec((BN, HID), lambda i: (i, 0)),
            pl.BlockSpec((L, BN), lambda i: (0, i)),
            pl.BlockSpec((L, HID), lambda i: (0, 0)),
            pl.BlockSpec((HID, DO), lambda i: (0, 0)),
            pl.BlockSpec((DO,), lambda i: (0,)),
        ],
        out_specs=pl.BlockSpec((BN, DO), lambda i: (i, 0)),
        out_shape=jax.ShapeDtypeStruct((NP, DO), jnp.float32),
    )(p, xls, dis, conv_bias, dec_W, dec_b)


def kernel(x, edge_index, edge_attr, emb_W, emb_b, ln_g, ln_b, lin_W,
           mlp_W1, mlp_b1, mlp_W2, mlp_b2, conv_bias, dec_W, dec_b):
    x2 = jnp.pad(x.reshape(N, -1), ((0, NP - N), (0, 0)))
    pad = E_PAD - E
    row_p = jnp.concatenate([edge_index[0], jnp.zeros((pad,), edge_index.dtype)])
    col_p = jnp.concatenate([edge_index[1], jnp.zeros((pad,), edge_index.dtype)])

    ea_p = jnp.concatenate([edge_attr, jnp.zeros((pad, EDGE_DIM), edge_attr.dtype)])
    w = _edge_mlp(ea_p, mlp_W1, mlp_b1, mlp_W2, mlp_b2)
    col3 = col_p.reshape(NW, NCHUNK, C)
    wls = [w[l].reshape(NW, NCHUNK, C) for l in range(L)]
    degp = _deg_partials(col3, wls)
    row4 = row_p.reshape(NW, NSEG, SEG, C)
    col4 = col_p.reshape(NW, NSEG, SEG, C)
    wl4s = [wl.reshape(NW, NSEG, SEG, C) for wl in wls]
    dis, xls = _embed(x2, emb_W, emb_b, ln_g, ln_b, degp, lin_W[0])
    for l in range(L):
        p = _msg_pass(xls, row4, col4, wl4s[l])
        if l < L - 1:
            xls = _mid(p, xls, dis, conv_bias, lin_W, l)
        else:
            out = _final(p, xls, dis, conv_bias, dec_W, dec_b)
    return out[:N].reshape(N, OUT_FEAT, FH)


# speedup vs baseline: 1.1923x; 1.0169x over previous
import functools

import jax
import jax.numpy as jnp
from jax import lax
from jax.experimental import pallas as pl
from jax.experimental.pallas import tpu as pltpu
from jax.experimental.pallas import tpu_sc as plsc

N = 10000
NP = 10240
E = 320000
HID = 128
EDGE_DIM = 16
HID_E = 64
L = 5
FH = 12
OUT_FEAT = 4

NCORE = 2
NSUB = 16
NW = NCORE * NSUB
EW = 10240
E_PAD = NW * EW
C = 64
NCHUNK = EW // C
RPS = NP // NSUB

BE = 2048
BN = 512


def _edge_mlp_body(ea_ref, W1_ref, b1_ref, W2_ref, b2_ref, *w_refs):
    i = pl.program_id(0)
    ea = ea_ref[...]
    eid = i * BE + lax.broadcasted_iota(jnp.int32, (BE, 1), 0)
    valid = eid < E
    w_ref = w_refs[0]
    rows = []
    for l in range(L):
        t = jnp.maximum(ea @ W1_ref[l] + b1_ref[l][None, :], 0.0)
        s = t @ W2_ref[l] + b2_ref[l][None, :]
        rows.append(jnp.where(valid, jax.nn.sigmoid(s), 0.0)[:, 0][None, :])
    w_ref[...] = jnp.concatenate(rows, axis=0)


def _edge_mlp(ea_p, W1, b1, W2, b2):
    grid = (E_PAD // BE,)
    return pl.pallas_call(
        _edge_mlp_body,
        grid=grid,
        in_specs=[
            pl.BlockSpec((BE, EDGE_DIM), lambda i: (i, 0)),
            pl.BlockSpec((L, EDGE_DIM, HID_E), lambda i: (0, 0, 0)),
            pl.BlockSpec((L, HID_E), lambda i: (0, 0)),
            pl.BlockSpec((L, HID_E, 1), lambda i: (0, 0, 0)),
            pl.BlockSpec((L, 1), lambda i: (0, 0)),
        ],
        out_specs=pl.BlockSpec((L, BE), lambda i: (0, i)),
        out_shape=jax.ShapeDtypeStruct((L, E_PAD), jnp.float32),
    )(ea_p, W1, b1, W2, b2)


_DWIN = 8


def _deg_body(z_hbm, col_hbm, w0_h, w1_h, w2_h, w3_h, w4_h,
              o0, o1, o2, o3, o4,
              a0, a1, a2, a3, a4, colbuf, wb0, wb1, wb2, wb3, wb4, sem):
    ci = lax.axis_index("c")
    si = lax.axis_index("s")
    wid = ci * NSUB + si
    accs = (a0, a1, a2, a3, a4)
    wbufs = (wb0, wb1, wb2, wb3, wb4)
    whs = (w0_h, w1_h, w2_h, w3_h, w4_h)
    for l in range(L):
        pltpu.sync_copy(z_hbm, accs[l].at[pl.ds(si * RPS, RPS)])
        pltpu.sync_copy(whs[l].at[wid], wbufs[l])
    pltpu.sync_copy(col_hbm.at[wid], colbuf)
    plsc.subcore_barrier()

    def s_desc(k, l):
        return pltpu.make_async_copy(
            wbufs[l].at[k], accs[l].at[colbuf.at[k]], sem)

    for k in range(_DWIN):
        for l in range(L):
            s_desc(k, l).start(add=True)

    def chunk_body(k, _):
        for l in range(L):
            s_desc(k - _DWIN, l).wait()
            s_desc(k, l).start(add=True)
        return 0

    lax.fori_loop(_DWIN, NCHUNK, chunk_body, 0)
    for k in range(_DWIN):
        for l in range(L):
            s_desc(NCHUNK - _DWIN + k, l).wait()
    plsc.subcore_barrier()
    outs = (o0, o1, o2, o3, o4)
    for l in range(L):
        pltpu.sync_copy(accs[l].at[pl.ds(si * RPS, RPS)],
                        outs[l].at[ci, pl.ds(si * RPS, RPS)])


def _deg_partials(col3, wls):
    mesh = plsc.VectorSubcoreMesh(core_axis_name="c", subcore_axis_name="s")
    f = pl.kernel(
        _deg_body,
        out_type=[jax.ShapeDtypeStruct((NCORE, NP), jnp.float32)
                  for _ in range(L)],
        mesh=mesh,
        scratch_types=(
            [pltpu.VMEM_SHARED((NP,), jnp.float32) for _ in range(L)]
            + [pltpu.VMEM((NCHUNK, C), jnp.int32)]
            + [pltpu.VMEM((NCHUNK, C), jnp.float32) for _ in range(L)]
            + [pltpu.SemaphoreType.DMA]
        ),
    )
    return f(jnp.zeros((RPS,), jnp.float32), col3, *wls)


def _embed_body(x_ref, eW_ref, eb_ref, g_ref, b_ref,
                d0_ref, d1_ref, d2_ref, d3_ref, d4_ref, W0_ref,
                dis_ref, xls_ref):
    h = x_ref[...] @ eW_ref[...] + eb_ref[...][None, :]
    mu = jnp.mean(h, axis=-1, keepdims=True)
    var = jnp.mean((h - mu) ** 2, axis=-1, keepdims=True)
    h = (h - mu) * lax.rsqrt(var + 1e-5) * g_ref[...][None, :] + b_ref[...][None, :]
    h = jnp.maximum(h, 0.0)
    drefs = (d0_ref, d1_ref, d2_ref, d3_ref, d4_ref)
    dis = jnp.concatenate(
        [lax.rsqrt(1.0 + dr[0] + dr[1])[None, :] for dr in drefs], axis=0)
    dis_ref[...] = dis
    xls_ref[...] = dis[0][:, None] * (h @ W0_ref[...])


def _embed(x2, emb_W, emb_b, ln_g, ln_b, degp, lin_W0):
    grid = (NP // BN,)
    return pl.pallas_call(
        _embed_body,
        grid=grid,
        in_specs=[
            pl.BlockSpec((BN, EDGE_DIM * 3), lambda i: (i, 0)),
            pl.BlockSpec((EDGE_DIM * 3, HID), lambda i: (0, 0)),
            pl.BlockSpec((HID,), lambda i: (0,)),
            pl.BlockSpec((HID,), lambda i: (0,)),
            pl.BlockSpec((HID,), lambda i: (0,)),
        ] + [pl.BlockSpec((NCORE, BN), lambda i: (0, i)) for _ in range(L)] + [
            pl.BlockSpec((HID, HID), lambda i: (0, 0)),
        ],
        out_specs=[
            pl.BlockSpec((L, BN), lambda i: (0, i)),
            pl.BlockSpec((BN, HID), lambda i: (i, 0)),
        ],
        out_shape=[
            jax.ShapeDtypeStruct((L, NP), jnp.float32),
            jax.ShapeDtypeStruct((NP, HID), jnp.float32),
        ],
    )(x2, emb_W, emb_b, ln_g, ln_b, *degp, lin_W0)


NB = 4
SEG = 32
NSEG = NCHUNK // SEG


def _msg_body(xls_hbm, row4, col4, w4, out_hbm,
              acc, b0, b1, b2, b3, rowbuf, colbuf, wbuf,
              sg0, sg1, sg2, sg3, ss0, ss1, ss2, ss3):
    ci = lax.axis_index("c")
    si = lax.axis_index("s")
    wid = ci * NSUB + si
    bufs = (b0, b1, b2, b3)
    sgs = (sg0, sg1, sg2, sg3)
    sss = (ss0, ss1, ss2, ss3)
    zeros = jnp.zeros((16,), jnp.float32)

    def zrow(i, _):
        for j in range(HID // 16):
            b0[i, pl.ds(j * 16, 16)] = zeros
        return 0

    lax.fori_loop(0, C, zrow, 0)
    for k in range(RPS // C):
        pltpu.sync_copy(b0, acc.at[pl.ds(si * RPS + k * C, C)])
    plsc.subcore_barrier()

    def g_desc(j, i):
        return pltpu.make_async_copy(xls_hbm.at[rowbuf.at[j]], bufs[i], sgs[i])

    def s_desc(j, i):
        return pltpu.make_async_copy(bufs[i], acc.at[colbuf.at[j]], sss[i])

    def scale(j, i):
        buf = bufs[i]

        def g(t, _):
            wv = wbuf[j, pl.ds(t * 16, 16)]
            for kk in range(16):
                we = wv[kk]
                e = t * 16 + kk
                for jj in range(HID // 16):
                    sl = pl.ds(jj * 16, 16)
                    buf[e, sl] = buf[e, sl] * we
            return 0

        lax.fori_loop(0, C // 16, g, 0)

    def seg_body(s, _):
        pltpu.sync_copy(row4.at[wid, s], rowbuf)
        pltpu.sync_copy(col4.at[wid, s], colbuf)
        pltpu.sync_copy(w4.at[wid, s], wbuf)
        for i in range(NB):
            g_desc(i, i).start()

        def body(k2, _):
            js = [k2 * NB + i for i in range(NB)]
            for i in range(NB):
                g_desc(js[i], i).wait()
                scale(js[i], i)
                s_desc(js[i], i).start(add=True)
            for i in range(NB):
                s_desc(js[i], i).wait()

                @pl.when(js[i] + NB < SEG)
                def _():
                    g_desc(js[i] + NB, i).start()
            return 0

        lax.fori_loop(0, SEG // NB, body, 0)
        return 0

    lax.fori_loop(0, NSEG, seg_body, 0)
    plsc.subcore_barrier()
    pltpu.sync_copy(acc.at[pl.ds(si * RPS, RPS)],
                    out_hbm.at[ci, pl.ds(si * RPS, RPS)])


def _msg_pass(xls, row4, col4, wl4):
    mesh = plsc.VectorSubcoreMesh(core_axis_name="c", subcore_axis_name="s")
    f = pl.kernel(
        _msg_body,
        out_type=jax.ShapeDtypeStruct((NCORE, NP, HID), jnp.float32),
        mesh=mesh,
        scratch_types=(
            [pltpu.VMEM_SHARED((NP, HID), jnp.float32)]
            + [pltpu.VMEM((C, HID), jnp.float32) for _ in range(NB)]
            + [pltpu.VMEM((SEG, C), jnp.int32)] * 2
            + [pltpu.VMEM((SEG, C), jnp.float32)]
            + [pltpu.SemaphoreType.DMA] * (2 * NB)
        ),
    )
    return f(xls, row4, col4, wl4)


def _mid_body(p_ref, xls_ref, dis_ref, bias_ref, W_ref, out_ref, *, l):
    ssum = p_ref[0] + p_ref[1] + xls_ref[...]
    dis = dis_ref[...]
    h = jnp.maximum(dis[l][:, None] * ssum + bias_ref[l][None, :], 0.0)
    out_ref[...] = dis[l + 1][:, None] * (h @ W_ref[0])


def _mid(p, xls, dis, conv_bias, lin_W, l):
    grid = (NP // BN,)
    return pl.pallas_call(
        functools.partial(_mid_body, l=l),
        grid=grid,
        in_specs=[
            pl.BlockSpec((NCORE, BN, HID), lambda i: (0, i, 0)),
            pl.BlockSpec((BN, HID), lambda i: (i, 0)),
            pl.BlockSpec((L, BN), lambda i: (0, i)),
            pl.BlockSpec((L, HID), lambda i: (0, 0)),
            pl.BlockSpec((1, HID, HID), lambda i: (l + 1, 0, 0)),
        ],
        out_specs=pl.BlockSpec((BN, HID), lambda i: (i, 0)),
        out_shape=jax.ShapeDtypeStruct((NP, HID), jnp.float32),
    )(p, xls, dis, conv_bias, lin_W)


def _final_body(p_ref, xls_ref, dis_ref, bias_ref, dW_ref, db_ref, out_ref):
    ssum = p_ref[0] + p_ref[1] + xls_ref[...]
    dis = dis_ref[...]
    h = jnp.maximum(dis[L - 1][:, None] * ssum + bias_ref[L - 1][None, :], 0.0)
    out_ref[...] = h @ dW_ref[...] + db_ref[...][None, :]


def _final(p, xls, dis, conv_bias, dec_W, dec_b):
    grid = (NP // BN,)
    DO = OUT_FEAT * FH
    return pl.pallas_call(
        _final_body,
        grid=grid,
        in_specs=[
            pl.BlockSpec((NCORE, BN, HID), lambda i: (0, i, 0)),
            pl.BlockSpec((BN, HID), lambda i: (i, 0)),
            pl.BlockSpec((L, BN), lambda i: (0, i)),
            pl.BlockSpec((L, HID), lambda i: (0, 0)),
            pl.BlockSpec((HID, DO), lambda i: (0, 0)),
            pl.BlockSpec((DO,), lambda i: (0,)),
        ],
        out_specs=pl.BlockSpec((BN, DO), lambda i: (i, 0)),
        out_shape=jax.ShapeDtypeStruct((NP, DO), jnp.float32),
    )(p, xls, dis, conv_bias, dec_W, dec_b)


def kernel(x, edge_index, edge_attr, emb_W, emb_b, ln_g, ln_b, lin_W,
           mlp_W1, mlp_b1, mlp_W2, mlp_b2, conv_bias, dec_W, dec_b):
    x2 = jnp.pad(x.reshape(N, -1), ((0, NP - N), (0, 0)))
    pad = E_PAD - E
    row_p = jnp.concatenate([edge_index[0], jnp.zeros((pad,), edge_index.dtype)])
    col_p = jnp.concatenate([edge_index[1], jnp.zeros((pad,), edge_index.dtype)])

    ea_p = jnp.concatenate([edge_attr, jnp.zeros((pad, EDGE_DIM), edge_attr.dtype)])
    w = _edge_mlp(ea_p, mlp_W1, mlp_b1, mlp_W2, mlp_b2)
    col3 = col_p.reshape(NW, NCHUNK, C)
    wls = [w[l].reshape(NW, NCHUNK, C) for l in range(L)]
    degp = _deg_partials(col3, wls)
    row4 = row_p.reshape(NW, NSEG, SEG, C)
    col4 = col_p.reshape(NW, NSEG, SEG, C)
    wl4s = [wl.reshape(NW, NSEG, SEG, C) for wl in wls]
    dis, xls = _embed(x2, emb_W, emb_b, ln_g, ln_b, degp, lin_W[0])
    for l in range(L):
        p = _msg_pass(xls, row4, col4, wl4s[l])
        if l < L - 1:
            xls = _mid(p, xls, dis, conv_bias, lin_W, l)
        else:
            out = _final(p, xls, dis, conv_bias, dec_W, dec_b)
    return out[:N].reshape(N, OUT_FEAT, FH)
